# baseline (speedup 1.0000x reference)
import functools

import jax
import jax.numpy as jnp
from jax import lax
from jax.experimental import pallas as pl
from jax.experimental.pallas import tpu as pltpu
from jax.experimental.pallas import tpu_sc as plsc

B, T, H = 16, 2048, 1024
B_SC = 8
C_SC = 512
NQ = 4
HH = C_SC // NQ
R = 64
NCHUNK = T // R
L = 16
GV = HH // L
UNROLL = 4

_mesh = plsc.VectorSubcoreMesh(core_axis_name="c", subcore_axis_name="s")


@functools.partial(
    pl.kernel,
    out_type=jax.ShapeDtypeStruct((B_SC, 2 * C_SC), jnp.float32),
    mesh=_mesh,
    compiler_params=pltpu.CompilerParams(needs_layout_passes=False),
    scratch_types=[
        pltpu.VMEM((R, HH), jnp.float32),
        pltpu.VMEM((R, HH), jnp.float32),
        pltpu.VMEM((R,), jnp.float32),
        pltpu.VMEM((R,), jnp.float32),
        pltpu.VMEM((2, HH), jnp.float32),
        pltpu.VMEM((L,), jnp.int32),
        pltpu.VMEM((HH,), jnp.float32),
        pltpu.SemaphoreType.DMA,
        pltpu.SemaphoreType.DMA,
    ],
)
def _pool_sc(x_hbm, wl_hbm, len_hbm, out_hbm, xbuf0, xbuf1, wlbuf0, wlbuf1,
             accbuf, lenbuf, outstage, sem0, sem1):
    wid = lax.axis_index("s") * 2 + lax.axis_index("c")
    b = wid // NQ
    c0 = (wid % NQ) * HH

    bufs = ((xbuf0, wlbuf0, sem0), (xbuf1, wlbuf1, sem1))

    def x_copy(ci, xb, sem):
        return pltpu.make_async_copy(
            x_hbm.at[b, pl.ds(ci * R, R), pl.ds(c0, HH)], xb, sem)

    def wl_copy(ci, wlb, sem):
        return pltpu.make_async_copy(wl_hbm.at[b, pl.ds(ci * R, R)], wlb, sem)

    pltpu.sync_copy(len_hbm, lenbuf)
    lenvi = plsc.load_gather(lenbuf, [jnp.full((L,), b, jnp.int32)])
    len_s = jnp.max(lenvi)

    zeros = jnp.zeros((L,), jnp.float32)
    ones = jnp.ones((L,), jnp.float32)
    for v in range(2):
        for j in range(GV):
            accbuf[v, pl.ds(j * L, L)] = zeros

    for par in range(2):
        xb, wlb, sem = bufs[par]
        x_copy(par, xb, sem).start()
        wl_copy(par, wlb, sem).start()

    def chunk2_body(k, cnt):
        for par in range(2):
            ci = 2 * k + par
            xb, wlb, sem = bufs[par]
            t0 = ci * R
            x_copy(ci, xb, sem).wait()
            wl_copy(ci, wlb, sem).wait()

            for q in range(R // L):
                cnt = cnt + wlb[pl.ds(q * L, L)]

            @pl.when(t0 >= len_s)
            def _():
                accs = tuple(accbuf[1, pl.ds(j * L, L)] for j in range(GV))

                def row_body(r2, carry):
                    out = list(carry)
                    for dr in range(UNROLL):
                        r = r2 * UNROLL + dr
                        wlv = plsc.load_gather(
                            wlb, [jnp.full((L,), r, jnp.int32)])
                        for j in range(GV):
                            xv = xb[r, pl.ds(j * L, L)]
                            out[j] = out[j] + wlv * xv
                    return tuple(out)

                accs = lax.fori_loop(0, R // UNROLL, row_body, accs)
                for j in range(GV):
                    accbuf[1, pl.ds(j * L, L)] = accs[j]

            @pl.when(t0 + R <= len_s)
            def _():
                accs = tuple(accbuf[v, pl.ds(j * L, L)]
                             for v in range(2) for j in range(GV))

                def row_body(r2, carry):
                    out = list(carry)
                    for dr in range(UNROLL):
                        r = r2 * UNROLL + dr
                        wlv = plsc.load_gather(
                            wlb, [jnp.full((L,), r, jnp.int32)])
                        for j in range(GV):
                            xv = xb[r, pl.ds(j * L, L)]
                            out[j] = out[j] + xv
                            out[GV + j] = out[GV + j] + wlv * xv
                    return tuple(out)

                accs = lax.fori_loop(0, R // UNROLL, row_body, accs)
                for v in range(2):
                    for j in range(GV):
                        accbuf[v, pl.ds(j * L, L)] = accs[v * GV + j]

            @pl.when(jnp.logical_and(t0 < len_s, t0 + R > len_s))
            def _():
                accs = tuple(accbuf[v, pl.ds(j * L, L)]
                             for v in range(2) for j in range(GV))

                def row_body(r2, carry):
                    out = list(carry)
                    for dr in range(UNROLL):
                        r = r2 * UNROLL + dr
                        wlv = plsc.load_gather(
                            wlb, [jnp.full((L,), r, jnp.int32)])
                        rgv = jnp.full((L,), t0 + r, jnp.int32)
                        wgv = jnp.where(rgv < lenvi, ones, zeros)
                        for j in range(GV):
                            xv = xb[r, pl.ds(j * L, L)]
                            out[j] = out[j] + wgv * xv
                            out[GV + j] = out[GV + j] + wlv * xv
                    return tuple(out)

                accs = lax.fori_loop(0, R // UNROLL, row_body, accs)
                for v in range(2):
                    for j in range(GV):
                        accbuf[v, pl.ds(j * L, L)] = accs[v * GV + j]

            @pl.when(ci + 2 < NCHUNK)
            def _():
                x_copy(ci + 2, xb, sem).start()
                wl_copy(ci + 2, wlb, sem).start()
        return cnt

    cnt = lax.fori_loop(0, NCHUNK // 2, chunk2_body, zeros)

    deng = jnp.maximum(lenvi.astype(jnp.float32), ones)
    denl = jnp.maximum(jnp.full((L,), jnp.sum(cnt)), ones)
    for v, den in ((0, deng), (1, denl)):
        for j in range(GV):
            outstage[pl.ds(j * L, L)] = accbuf[v, pl.ds(j * L, L)] / den
        pltpu.sync_copy(outstage.at[pl.ds(0, HH)],
                        out_hbm.at[b, pl.ds(v * C_SC + c0, HH)])


TBLK = 256
NT = T // TBLK


def _tc_body(x_ref, wg_ref, wl_ref, out_ref, acc_ref, den_ref):
    t = pl.program_id(1)
    w = out_ref.shape[2]

    @pl.when(t == 0)
    def _():
        acc_ref[...] = jnp.zeros_like(acc_ref)
        den_ref[0] = 0.0
        den_ref[1] = 0.0

    x = x_ref[0]
    wg = wg_ref[0, 0]
    wl = wl_ref[0, 0]
    acc_ref[0:1] += jnp.sum(x * wg, axis=0, keepdims=True)
    acc_ref[1:2] += jnp.sum(x * wl, axis=0, keepdims=True)
    den_ref[0] += jnp.sum(wg)
    den_ref[1] += jnp.sum(wl)

    @pl.when(t == NT - 1)
    def _():
        deng = jnp.maximum(den_ref[0], 1.0)
        denl = jnp.maximum(den_ref[1], 1.0)
        out_ref[...] = jnp.concatenate(
            [acc_ref[0:1] / deng, acc_ref[1:2] / denl], axis=0
        ).reshape(1, 2, w)


def _make_tc(nb, b_off, w, c_off):
    return pl.pallas_call(
        _tc_body,
        grid=(nb, NT),
        in_specs=[
            pl.BlockSpec((1, TBLK, w), lambda bb, t: (bb + b_off, t, c_off // w)),
            pl.BlockSpec((1, 1, TBLK, 1), lambda bb, t: (bb + b_off, t, 0, 0)),
            pl.BlockSpec((1, 1, TBLK, 1), lambda bb, t: (bb + b_off, t, 0, 0)),
        ],
        out_specs=pl.BlockSpec((1, 2, w), lambda bb, t: (bb, 0, 0)),
        out_shape=jax.ShapeDtypeStruct((nb, 2, w), jnp.float32),
        scratch_shapes=[
            pltpu.VMEM((2, w), jnp.float32),
            pltpu.SMEM((2,), jnp.float32),
        ],
        compiler_params=pltpu.CompilerParams(
            dimension_semantics=("parallel", "arbitrary")),
    )


_pool_tc_hi = _make_tc(B - B_SC, B_SC, H, 0)
_pool_tc_lo = _make_tc(B_SC, 0, H - C_SC, C_SC)


def kernel(inputs, lengths, mask):
    wl = mask.astype(jnp.float32)
    lens = lengths.astype(jnp.int32)
    wg = (jnp.arange(T, dtype=jnp.int32)[None, :]
          < lens[:, None]).astype(jnp.float32)
    wg4 = wg.reshape(B, NT, TBLK, 1)
    wl4 = wl.reshape(B, NT, TBLK, 1)
    out_sc = _pool_sc(inputs, wl, lens)
    out_hi = _pool_tc_hi(inputs, wg4, wl4)
    out_lo = _pool_tc_lo(inputs, wg4, wl4)
    rows_lo = jnp.concatenate(
        [out_sc.reshape(B_SC, 2, C_SC), out_lo], axis=2).reshape(B_SC, 2 * H)
    rows_hi = out_hi.reshape(B - B_SC, 2 * H)
    return jnp.concatenate([rows_lo, rows_hi], axis=0)


# speedup vs baseline: 1.2239x; 1.2239x over previous
import functools

import jax
import jax.numpy as jnp
from jax import lax
from jax.experimental import pallas as pl
from jax.experimental.pallas import tpu as pltpu
from jax.experimental.pallas import tpu_sc as plsc

B, T, H = 16, 2048, 1024
B_SC = 8
C_SC = 512
NQ = 4
HH = C_SC // NQ
R = 64
NCHUNK = T // R
L = 16
GV = HH // L
UNROLL = 4

_mesh = plsc.VectorSubcoreMesh(core_axis_name="c", subcore_axis_name="s")


@functools.partial(
    pl.kernel,
    out_type=jax.ShapeDtypeStruct((B_SC, 2 * C_SC), jnp.float32),
    mesh=_mesh,
    compiler_params=pltpu.CompilerParams(needs_layout_passes=False),
    scratch_types=[
        pltpu.VMEM((R, HH), jnp.float32),
        pltpu.VMEM((R, HH), jnp.float32),
        pltpu.VMEM((R,), jnp.float32),
        pltpu.VMEM((R,), jnp.float32),
        pltpu.VMEM((2, HH), jnp.float32),
        pltpu.VMEM((L,), jnp.int32),
        pltpu.VMEM((HH,), jnp.float32),
        pltpu.SemaphoreType.DMA,
        pltpu.SemaphoreType.DMA,
    ],
)
def _pool_sc(x_hbm, wl_hbm, len_hbm, out_hbm, xbuf0, xbuf1, wlbuf0, wlbuf1,
             accbuf, lenbuf, outstage, sem0, sem1):
    wid = lax.axis_index("s") * 2 + lax.axis_index("c")
    b = wid // NQ
    c0 = (wid % NQ) * HH

    bufs = ((xbuf0, wlbuf0, sem0), (xbuf1, wlbuf1, sem1))

    def x_copy(ci, xb, sem):
        return pltpu.make_async_copy(
            x_hbm.at[b, pl.ds(ci * R, R), pl.ds(c0, HH)], xb, sem)

    def wl_copy(ci, wlb, sem):
        return pltpu.make_async_copy(wl_hbm.at[b, pl.ds(ci * R, R)], wlb, sem)

    pltpu.sync_copy(len_hbm, lenbuf)
    lenvi = plsc.load_gather(lenbuf, [jnp.full((L,), b, jnp.int32)])
    len_s = jnp.max(lenvi)

    zeros = jnp.zeros((L,), jnp.float32)
    ones = jnp.ones((L,), jnp.float32)
    for v in range(2):
        for j in range(GV):
            accbuf[v, pl.ds(j * L, L)] = zeros

    for par in range(2):
        xb, wlb, sem = bufs[par]
        x_copy(par, xb, sem).start()
        wl_copy(par, wlb, sem).start()

    def chunk2_body(k, cnt):
        for par in range(2):
            ci = 2 * k + par
            xb, wlb, sem = bufs[par]
            t0 = ci * R
            x_copy(ci, xb, sem).wait()
            wl_copy(ci, wlb, sem).wait()

            for q in range(R // L):
                cnt = cnt + wlb[pl.ds(q * L, L)]

            @pl.when(t0 >= len_s)
            def _():
                accs = tuple(accbuf[1, pl.ds(j * L, L)] for j in range(GV))

                def row_body(r2, carry):
                    out = list(carry)
                    for dr in range(UNROLL):
                        r = r2 * UNROLL + dr
                        wlv = plsc.load_gather(
                            wlb, [jnp.full((L,), r, jnp.int32)])
                        for j in range(GV):
                            xv = xb[r, pl.ds(j * L, L)]
                            out[j] = out[j] + wlv * xv
                    return tuple(out)

                accs = lax.fori_loop(0, R // UNROLL, row_body, accs)
                for j in range(GV):
                    accbuf[1, pl.ds(j * L, L)] = accs[j]

            @pl.when(t0 + R <= len_s)
            def _():
                accs = tuple(accbuf[v, pl.ds(j * L, L)]
                             for v in range(2) for j in range(GV))

                def row_body(r2, carry):
                    out = list(carry)
                    for dr in range(UNROLL):
                        r = r2 * UNROLL + dr
                        wlv = plsc.load_gather(
                            wlb, [jnp.full((L,), r, jnp.int32)])
                        for j in range(GV):
                            xv = xb[r, pl.ds(j * L, L)]
                            out[j] = out[j] + xv
                            out[GV + j] = out[GV + j] + wlv * xv
                    return tuple(out)

                accs = lax.fori_loop(0, R // UNROLL, row_body, accs)
                for v in range(2):
                    for j in range(GV):
                        accbuf[v, pl.ds(j * L, L)] = accs[v * GV + j]

            @pl.when(jnp.logical_and(t0 < len_s, t0 + R > len_s))
            def _():
                accs = tuple(accbuf[v, pl.ds(j * L, L)]
                             for v in range(2) for j in range(GV))

                def row_body(r2, carry):
                    out = list(carry)
                    for dr in range(UNROLL):
                        r = r2 * UNROLL + dr
                        wlv = plsc.load_gather(
                            wlb, [jnp.full((L,), r, jnp.int32)])
                        rgv = jnp.full((L,), t0 + r, jnp.int32)
                        wgv = jnp.where(rgv < lenvi, ones, zeros)
                        for j in range(GV):
                            xv = xb[r, pl.ds(j * L, L)]
                            out[j] = out[j] + wgv * xv
                            out[GV + j] = out[GV + j] + wlv * xv
                    return tuple(out)

                accs = lax.fori_loop(0, R // UNROLL, row_body, accs)
                for v in range(2):
                    for j in range(GV):
                        accbuf[v, pl.ds(j * L, L)] = accs[v * GV + j]

            @pl.when(ci + 2 < NCHUNK)
            def _():
                x_copy(ci + 2, xb, sem).start()
                wl_copy(ci + 2, wlb, sem).start()
        return cnt

    cnt = lax.fori_loop(0, NCHUNK // 2, chunk2_body, zeros)

    deng = jnp.maximum(lenvi.astype(jnp.float32), ones)
    denl = jnp.maximum(jnp.full((L,), jnp.sum(cnt)), ones)
    for v, den in ((0, deng), (1, denl)):
        for j in range(GV):
            outstage[pl.ds(j * L, L)] = accbuf[v, pl.ds(j * L, L)] / den
        pltpu.sync_copy(outstage.at[pl.ds(0, HH)],
                        out_hbm.at[b, pl.ds(v * C_SC + c0, HH)])


TBLK = 256
NT = T // TBLK


def _tc_body(x_ref, wg_ref, wl_ref, out_ref, acc_ref, den_ref):
    t = pl.program_id(1)
    w = out_ref.shape[2]

    @pl.when(t == 0)
    def _():
        acc_ref[...] = jnp.zeros_like(acc_ref)
        den_ref[0] = 0.0
        den_ref[1] = 0.0

    x = x_ref[0]
    wg = jnp.transpose(wg_ref[0, 0])
    wl = jnp.transpose(wl_ref[0, 0])
    acc_ref[0:1] += jnp.sum(x * wg, axis=0, keepdims=True)
    acc_ref[1:2] += jnp.sum(x * wl, axis=0, keepdims=True)
    den_ref[0] += jnp.sum(wg)
    den_ref[1] += jnp.sum(wl)

    @pl.when(t == NT - 1)
    def _():
        deng = jnp.maximum(den_ref[0], 1.0)
        denl = jnp.maximum(den_ref[1], 1.0)
        out_ref[...] = jnp.concatenate(
            [acc_ref[0:1] / deng, acc_ref[1:2] / denl], axis=0
        ).reshape(1, 2, w)


def _make_tc(nb, b_off, w, c_off):
    return pl.pallas_call(
        _tc_body,
        grid=(nb, NT),
        in_specs=[
            pl.BlockSpec((1, TBLK, w), lambda bb, t: (bb + b_off, t, c_off // w)),
            pl.BlockSpec((1, 1, 1, TBLK), lambda bb, t: (bb + b_off, t, 0, 0)),
            pl.BlockSpec((1, 1, 1, TBLK), lambda bb, t: (bb + b_off, t, 0, 0)),
        ],
        out_specs=pl.BlockSpec((1, 2, w), lambda bb, t: (bb, 0, 0)),
        out_shape=jax.ShapeDtypeStruct((nb, 2, w), jnp.float32),
        scratch_shapes=[
            pltpu.VMEM((2, w), jnp.float32),
            pltpu.SMEM((2,), jnp.float32),
        ],
        compiler_params=pltpu.CompilerParams(
            dimension_semantics=("parallel", "arbitrary")),
    )


_pool_tc_hi = _make_tc(B - B_SC, B_SC, H, 0)
_pool_tc_lo = _make_tc(B_SC, 0, H - C_SC, C_SC)


def kernel(inputs, lengths, mask):
    wl = mask.astype(jnp.float32)
    lens = lengths.astype(jnp.int32)
    wg = (jnp.arange(T, dtype=jnp.int32)[None, :]
          < lens[:, None]).astype(jnp.float32)
    wg4 = wg.reshape(B, NT, 1, TBLK)
    wl4 = wl.reshape(B, NT, 1, TBLK)
    out_sc = _pool_sc(inputs, wl, lens)
    out_hi = _pool_tc_hi(inputs, wg4, wl4)
    out_lo = _pool_tc_lo(inputs, wg4, wl4)
    rows_lo = jnp.concatenate(
        [out_sc.reshape(B_SC, 2, C_SC), out_lo], axis=2).reshape(B_SC, 2 * H)
    rows_hi = out_hi.reshape(B - B_SC, 2 * H)
    return jnp.concatenate([rows_lo, rows_hi], axis=0)


# speedup vs baseline: 1.8080x; 1.4773x over previous
import functools

import jax
import jax.numpy as jnp
from jax import lax
from jax.experimental import pallas as pl
from jax.experimental.pallas import tpu as pltpu
from jax.experimental.pallas import tpu_sc as plsc

B, T, H = 16, 2048, 1024
B_SC = 8
NQ = 4
HH = H // NQ
R = 64
NCHUNK = T // R
L = 16
GV = HH // L
UNROLL = 4
TBLK = 256
NT = T // TBLK

_mesh = plsc.VectorSubcoreMesh(core_axis_name="c", subcore_axis_name="s")


@functools.partial(
    pl.kernel,
    out_type=jax.ShapeDtypeStruct((B_SC, 2 * H), jnp.float32),
    mesh=_mesh,
    compiler_params=pltpu.CompilerParams(needs_layout_passes=False),
    scratch_types=[
        pltpu.VMEM((R, HH), jnp.float32),
        pltpu.VMEM((R, HH), jnp.float32),
        pltpu.VMEM((R,), jnp.float32),
        pltpu.VMEM((R,), jnp.float32),
        pltpu.VMEM((2, HH), jnp.float32),
        pltpu.VMEM((L,), jnp.int32),
        pltpu.VMEM((HH,), jnp.float32),
        pltpu.SemaphoreType.DMA,
        pltpu.SemaphoreType.DMA,
    ],
)
def _pool_sc(x_hbm, w4_hbm, len_hbm, out_hbm, xbuf0, xbuf1, wlbuf0, wlbuf1,
             accbuf, lenbuf, outstage, sem0, sem1):
    wid = lax.axis_index("s") * 2 + lax.axis_index("c")
    b = wid // NQ
    c0 = (wid % NQ) * HH

    bufs = ((xbuf0, wlbuf0, sem0), (xbuf1, wlbuf1, sem1))

    def x_copy(ci, xb, sem):
        return pltpu.make_async_copy(
            x_hbm.at[b, pl.ds(ci * R, R), pl.ds(c0, HH)], xb, sem)

    def wl_copy(ci, wlb, sem):
        return pltpu.make_async_copy(
            w4_hbm.at[b, ci // (TBLK // R), 1,
                      pl.ds((ci % (TBLK // R)) * R, R)], wlb, sem)

    pltpu.sync_copy(len_hbm, lenbuf)
    lenvi = plsc.load_gather(lenbuf, [jnp.full((L,), b, jnp.int32)])
    len_s = jnp.max(lenvi)

    zeros = jnp.zeros((L,), jnp.float32)
    ones = jnp.ones((L,), jnp.float32)
    for v in range(2):
        for j in range(GV):
            accbuf[v, pl.ds(j * L, L)] = zeros

    for par in range(2):
        xb, wlb, sem = bufs[par]
        x_copy(par, xb, sem).start()
        wl_copy(par, wlb, sem).start()

    def chunk2_body(k, cnt):
        for par in range(2):
            ci = 2 * k + par
            xb, wlb, sem = bufs[par]
            t0 = ci * R
            x_copy(ci, xb, sem).wait()
            wl_copy(ci, wlb, sem).wait()

            for q in range(R // L):
                cnt = cnt + wlb[pl.ds(q * L, L)]

            @pl.when(t0 >= len_s)
            def _():
                accs = tuple(accbuf[1, pl.ds(j * L, L)] for j in range(GV))

                def row_body(r2, carry):
                    out = list(carry)
                    for dr in range(UNROLL):
                        r = r2 * UNROLL + dr
                        wlv = plsc.load_gather(
                            wlb, [jnp.full((L,), r, jnp.int32)])
                        for j in range(GV):
                            xv = xb[r, pl.ds(j * L, L)]
                            out[j] = out[j] + wlv * xv
                    return tuple(out)

                accs = lax.fori_loop(0, R // UNROLL, row_body, accs)
                for j in range(GV):
                    accbuf[1, pl.ds(j * L, L)] = accs[j]

            @pl.when(t0 + R <= len_s)
            def _():
                accs = tuple(accbuf[v, pl.ds(j * L, L)]
                             for v in range(2) for j in range(GV))

                def row_body(r2, carry):
                    out = list(carry)
                    for dr in range(UNROLL):
                        r = r2 * UNROLL + dr
                        wlv = plsc.load_gather(
                            wlb, [jnp.full((L,), r, jnp.int32)])
                        for j in range(GV):
                            xv = xb[r, pl.ds(j * L, L)]
                            out[j] = out[j] + xv
                            out[GV + j] = out[GV + j] + wlv * xv
                    return tuple(out)

                accs = lax.fori_loop(0, R // UNROLL, row_body, accs)
                for v in range(2):
                    for j in range(GV):
                        accbuf[v, pl.ds(j * L, L)] = accs[v * GV + j]

            @pl.when(jnp.logical_and(t0 < len_s, t0 + R > len_s))
            def _():
                accs = tuple(accbuf[v, pl.ds(j * L, L)]
                             for v in range(2) for j in range(GV))

                def row_body(r2, carry):
                    out = list(carry)
                    for dr in range(UNROLL):
                        r = r2 * UNROLL + dr
                        wlv = plsc.load_gather(
                            wlb, [jnp.full((L,), r, jnp.int32)])
                        rgv = jnp.full((L,), t0 + r, jnp.int32)
                        wgv = jnp.where(rgv < lenvi, ones, zeros)
                        for j in range(GV):
                            xv = xb[r, pl.ds(j * L, L)]
                            out[j] = out[j] + wgv * xv
                            out[GV + j] = out[GV + j] + wlv * xv
                    return tuple(out)

                accs = lax.fori_loop(0, R // UNROLL, row_body, accs)
                for v in range(2):
                    for j in range(GV):
                        accbuf[v, pl.ds(j * L, L)] = accs[v * GV + j]

            @pl.when(ci + 2 < NCHUNK)
            def _():
                x_copy(ci + 2, xb, sem).start()
                wl_copy(ci + 2, wlb, sem).start()
        return cnt

    cnt = lax.fori_loop(0, NCHUNK // 2, chunk2_body, zeros)

    deng = jnp.maximum(lenvi.astype(jnp.float32), ones)
    denl = jnp.maximum(jnp.full((L,), jnp.sum(cnt)), ones)
    for v, den in ((0, deng), (1, denl)):
        for j in range(GV):
            outstage[pl.ds(j * L, L)] = accbuf[v, pl.ds(j * L, L)] / den
        pltpu.sync_copy(outstage.at[pl.ds(0, HH)],
                        out_hbm.at[b, pl.ds(v * H + c0, HH)])


def _tc_body(x_ref, w_ref, out_ref, acc_ref, den_ref):
    t = pl.program_id(1)

    @pl.when(t == 0)
    def _():
        acc_ref[...] = jnp.zeros_like(acc_ref)
        den_ref[0] = 0.0
        den_ref[1] = 0.0

    x = x_ref[0]
    w2 = w_ref[0, 0]
    acc_ref[...] += jnp.dot(w2, x, preferred_element_type=jnp.float32,
                            precision=jax.lax.Precision.HIGHEST)
    den_ref[0] += jnp.sum(w2[0:1])
    den_ref[1] += jnp.sum(w2[1:2])

    @pl.when(t == NT - 1)
    def _():
        deng = jnp.maximum(den_ref[0], 1.0)
        denl = jnp.maximum(den_ref[1], 1.0)
        out_ref[...] = jnp.concatenate(
            [acc_ref[0:1] / deng, acc_ref[1:2] / denl], axis=1
        ).reshape(1, 1, 2 * H)


_pool_tc = pl.pallas_call(
    _tc_body,
    grid=(B - B_SC, NT),
    in_specs=[
        pl.BlockSpec((1, TBLK, H), lambda bb, t: (bb + B_SC, t, 0)),
        pl.BlockSpec((1, 1, 2, TBLK), lambda bb, t: (bb + B_SC, t, 0, 0)),
    ],
    out_specs=pl.BlockSpec((1, 1, 2 * H), lambda bb, t: (bb, 0, 0)),
    out_shape=jax.ShapeDtypeStruct((B - B_SC, 1, 2 * H), jnp.float32),
    scratch_shapes=[
        pltpu.VMEM((2, H), jnp.float32),
        pltpu.SMEM((2,), jnp.float32),
    ],
    compiler_params=pltpu.CompilerParams(
        dimension_semantics=("parallel", "arbitrary")),
)


def kernel(inputs, lengths, mask):
    lens = lengths.astype(jnp.int32)
    wg = (jnp.arange(T, dtype=jnp.int32)[None, :]
          < lens[:, None]).astype(jnp.float32)
    wl = mask.astype(jnp.float32)
    w4 = jnp.stack([wg.reshape(B, NT, TBLK), wl.reshape(B, NT, TBLK)], axis=2)
    out_tc = _pool_tc(inputs, w4).reshape(B - B_SC, 2 * H)
    out_sc = _pool_sc(inputs, w4, lens)
    return jnp.concatenate([out_sc, out_tc], axis=0)


# speedup vs baseline: 2.2806x; 1.2614x over previous
import functools

import jax
import jax.numpy as jnp
from jax import lax
from jax.experimental import pallas as pl
from jax.experimental.pallas import tpu as pltpu
from jax.experimental.pallas import tpu_sc as plsc

B, T, H = 16, 2048, 1024
B_SC = 8
C_SC = 1024
NQ = 4
HH = C_SC // NQ
R = 128
NCHUNK = T // R
L = 16
GV = HH // L
UNROLL = 4
TBLK = 512
NT = T // TBLK

_mesh = plsc.VectorSubcoreMesh(core_axis_name="c", subcore_axis_name="s")


@functools.partial(
    pl.kernel,
    out_type=jax.ShapeDtypeStruct((B_SC, 2 * C_SC), jnp.float32),
    mesh=_mesh,
    compiler_params=pltpu.CompilerParams(needs_layout_passes=False),
    scratch_types=[
        pltpu.VMEM((R, HH), jnp.float32),
        pltpu.VMEM((R, HH), jnp.float32),
        pltpu.VMEM((R,), jnp.float32),
        pltpu.VMEM((R,), jnp.float32),
        pltpu.VMEM((2, HH), jnp.float32),
        pltpu.VMEM((L,), jnp.int32),
        pltpu.VMEM((HH,), jnp.float32),
        pltpu.SemaphoreType.DMA,
        pltpu.SemaphoreType.DMA,
    ],
)
def _pool_sc(x_hbm, w4_hbm, len_hbm, out_hbm, xbuf0, xbuf1, wlbuf0, wlbuf1,
             accbuf, lenbuf, outstage, sem0, sem1):
    wid = lax.axis_index("s") * 2 + lax.axis_index("c")
    b = wid // NQ
    c0 = (wid % NQ) * HH

    bufs = ((xbuf0, wlbuf0, sem0), (xbuf1, wlbuf1, sem1))

    def x_copy(ci, xb, sem):
        return pltpu.make_async_copy(
            x_hbm.at[b, pl.ds(ci * R, R), pl.ds(c0, HH)], xb, sem)

    def wl_copy(ci, wlb, sem):
        return pltpu.make_async_copy(
            w4_hbm.at[b, ci // (TBLK // R), 1,
                      pl.ds((ci % (TBLK // R)) * R, R)], wlb, sem)

    pltpu.sync_copy(len_hbm, lenbuf)
    lenvi = plsc.load_gather(lenbuf, [jnp.full((L,), b, jnp.int32)])
    len_s = jnp.max(lenvi)

    zeros = jnp.zeros((L,), jnp.float32)
    ones = jnp.ones((L,), jnp.float32)
    for v in range(2):
        for j in range(GV):
            accbuf[v, pl.ds(j * L, L)] = zeros

    for par in range(2):
        xb, wlb, sem = bufs[par]
        x_copy(par, xb, sem).start()
        wl_copy(par, wlb, sem).start()

    def chunk2_body(k, cnt):
        for par in range(2):
            ci = 2 * k + par
            xb, wlb, sem = bufs[par]
            t0 = ci * R
            x_copy(ci, xb, sem).wait()
            wl_copy(ci, wlb, sem).wait()

            for q in range(R // L):
                cnt = cnt + wlb[pl.ds(q * L, L)]

            @pl.when(t0 >= len_s)
            def _():
                accs = tuple(accbuf[1, pl.ds(j * L, L)] for j in range(GV))

                def row_body(r2, carry):
                    out = list(carry)
                    for dr in range(UNROLL):
                        r = r2 * UNROLL + dr
                        wlv = plsc.load_gather(
                            wlb, [jnp.full((L,), r, jnp.int32)])
                        for j in range(GV):
                            xv = xb[r, pl.ds(j * L, L)]
                            out[j] = out[j] + wlv * xv
                    return tuple(out)

                accs = lax.fori_loop(0, R // UNROLL, row_body, accs)
                for j in range(GV):
                    accbuf[1, pl.ds(j * L, L)] = accs[j]

            @pl.when(t0 + R <= len_s)
            def _():
                accs = tuple(accbuf[v, pl.ds(j * L, L)]
                             for v in range(2) for j in range(GV))

                def row_body(r2, carry):
                    out = list(carry)
                    for dr in range(UNROLL):
                        r = r2 * UNROLL + dr
                        wlv = plsc.load_gather(
                            wlb, [jnp.full((L,), r, jnp.int32)])
                        for j in range(GV):
                            xv = xb[r, pl.ds(j * L, L)]
                            out[j] = out[j] + xv
                            out[GV + j] = out[GV + j] + wlv * xv
                    return tuple(out)

                accs = lax.fori_loop(0, R // UNROLL, row_body, accs)
                for v in range(2):
                    for j in range(GV):
                        accbuf[v, pl.ds(j * L, L)] = accs[v * GV + j]

            @pl.when(jnp.logical_and(t0 < len_s, t0 + R > len_s))
            def _():
                accs = tuple(accbuf[v, pl.ds(j * L, L)]
                             for v in range(2) for j in range(GV))

                def row_body(r2, carry):
                    out = list(carry)
                    for dr in range(UNROLL):
                        r = r2 * UNROLL + dr
                        wlv = plsc.load_gather(
                            wlb, [jnp.full((L,), r, jnp.int32)])
                        rgv = jnp.full((L,), t0 + r, jnp.int32)
                        wgv = jnp.where(rgv < lenvi, ones, zeros)
                        for j in range(GV):
                            xv = xb[r, pl.ds(j * L, L)]
                            out[j] = out[j] + wgv * xv
                            out[GV + j] = out[GV + j] + wlv * xv
                    return tuple(out)

                accs = lax.fori_loop(0, R // UNROLL, row_body, accs)
                for v in range(2):
                    for j in range(GV):
                        accbuf[v, pl.ds(j * L, L)] = accs[v * GV + j]

            @pl.when(ci + 2 < NCHUNK)
            def _():
                x_copy(ci + 2, xb, sem).start()
                wl_copy(ci + 2, wlb, sem).start()
        return cnt

    cnt = lax.fori_loop(0, NCHUNK // 2, chunk2_body, zeros)

    deng = jnp.maximum(lenvi.astype(jnp.float32), ones)
    denl = jnp.maximum(jnp.full((L,), jnp.sum(cnt)), ones)
    for v, den in ((0, deng), (1, denl)):
        for j in range(GV):
            outstage[pl.ds(j * L, L)] = accbuf[v, pl.ds(j * L, L)] / den
        pltpu.sync_copy(outstage.at[pl.ds(0, HH)],
                        out_hbm.at[b, pl.ds(v * C_SC + c0, HH)])


def _tc_body(x_ref, w_ref, out_ref, acc_ref, den_ref):
    t = pl.program_id(1)
    w = x_ref.shape[2]

    @pl.when(t == 0)
    def _():
        acc_ref[...] = jnp.zeros_like(acc_ref)
        den_ref[0] = 0.0
        den_ref[1] = 0.0

    x = x_ref[0]
    w2 = w_ref[0, 0]
    acc_ref[...] += jnp.dot(w2, x, preferred_element_type=jnp.float32)
    den_ref[0] += jnp.sum(w2[0:1])
    den_ref[1] += jnp.sum(w2[1:2])

    @pl.when(t == NT - 1)
    def _():
        deng = jnp.maximum(den_ref[0], 1.0)
        denl = jnp.maximum(den_ref[1], 1.0)
        out_ref[...] = jnp.concatenate(
            [acc_ref[0:1] / deng, acc_ref[1:2] / denl], axis=0
        ).reshape(1, 2, w)


def _make_tc(nb, b_off, w, c_off):
    return pl.pallas_call(
        _tc_body,
        grid=(nb, NT),
        in_specs=[
            pl.BlockSpec((1, TBLK, w),
                         lambda bb, t: (bb + b_off, t, c_off // w)),
            pl.BlockSpec((1, 1, 2, TBLK), lambda bb, t: (bb + b_off, t, 0, 0)),
        ],
        out_specs=pl.BlockSpec((1, 2, w), lambda bb, t: (bb, 0, 0)),
        out_shape=jax.ShapeDtypeStruct((nb, 2, w), jnp.float32),
        scratch_shapes=[
            pltpu.VMEM((2, w), jnp.float32),
            pltpu.SMEM((2,), jnp.float32),
        ],
        compiler_params=pltpu.CompilerParams(
            dimension_semantics=("parallel", "arbitrary")),
    )


_pool_tc_hi = _make_tc(B - B_SC, B_SC, H, 0)


def kernel(inputs, lengths, mask):
    lens = lengths.astype(jnp.int32)
    wg = (jnp.arange(T, dtype=jnp.int32)[None, :]
          < lens[:, None]).astype(jnp.float32)
    wl = mask.astype(jnp.float32)
    w4 = jnp.stack([wg.reshape(B, NT, TBLK), wl.reshape(B, NT, TBLK)], axis=2)
    out_hi = _pool_tc_hi(inputs, w4)
    out_sc = _pool_sc(inputs, w4, lens)
    rows_hi = out_hi.reshape(B - B_SC, 2 * H)
    return jnp.concatenate([out_sc, rows_hi], axis=0)


# speedup vs baseline: 2.3150x; 1.0150x over previous
import functools

import jax
import jax.numpy as jnp
from jax import lax
from jax.experimental import pallas as pl
from jax.experimental.pallas import tpu as pltpu
from jax.experimental.pallas import tpu_sc as plsc

B, T, H = 16, 2048, 1024
B_SC = 8
C_SC = 1024
NQ = 4
HH = C_SC // NQ
R = 128
NCHUNK = T // R
L = 16
GV = HH // L
UNROLL = 4
TBLK = 1024
NT = T // TBLK

_mesh = plsc.VectorSubcoreMesh(core_axis_name="c", subcore_axis_name="s")


@functools.partial(
    pl.kernel,
    out_type=jax.ShapeDtypeStruct((B_SC, 2 * C_SC), jnp.float32),
    mesh=_mesh,
    compiler_params=pltpu.CompilerParams(needs_layout_passes=False),
    scratch_types=[
        pltpu.VMEM((R, HH), jnp.float32),
        pltpu.VMEM((R, HH), jnp.float32),
        pltpu.VMEM((R,), jnp.float32),
        pltpu.VMEM((R,), jnp.float32),
        pltpu.VMEM((2, HH), jnp.float32),
        pltpu.VMEM((L,), jnp.int32),
        pltpu.VMEM((HH,), jnp.float32),
        pltpu.SemaphoreType.DMA,
        pltpu.SemaphoreType.DMA,
    ],
)
def _pool_sc(x_hbm, wl_hbm, len_hbm, out_hbm, xbuf0, xbuf1, wlbuf0, wlbuf1,
             accbuf, lenbuf, outstage, sem0, sem1):
    wid = lax.axis_index("s") * 2 + lax.axis_index("c")
    b = wid // NQ
    c0 = (wid % NQ) * HH

    bufs = ((xbuf0, wlbuf0, sem0), (xbuf1, wlbuf1, sem1))

    def x_copy(ci, xb, sem):
        return pltpu.make_async_copy(
            x_hbm.at[b, pl.ds(ci * R, R), pl.ds(c0, HH)], xb, sem)

    def wl_copy(ci, wlb, sem):
        return pltpu.make_async_copy(wl_hbm.at[b, pl.ds(ci * R, R)], wlb, sem)

    pltpu.sync_copy(len_hbm, lenbuf)
    lenvi = plsc.load_gather(lenbuf, [jnp.full((L,), b, jnp.int32)])
    len_s = jnp.max(lenvi)

    zeros = jnp.zeros((L,), jnp.float32)
    ones = jnp.ones((L,), jnp.float32)
    for v in range(2):
        for j in range(GV):
            accbuf[v, pl.ds(j * L, L)] = zeros

    for par in range(2):
        xb, wlb, sem = bufs[par]
        x_copy(par, xb, sem).start()
        wl_copy(par, wlb, sem).start()

    def chunk2_body(k, cnt):
        for par in range(2):
            ci = 2 * k + par
            xb, wlb, sem = bufs[par]
            t0 = ci * R
            x_copy(ci, xb, sem).wait()
            wl_copy(ci, wlb, sem).wait()

            for q in range(R // L):
                cnt = cnt + wlb[pl.ds(q * L, L)]

            @pl.when(t0 >= len_s)
            def _():
                accs = tuple(accbuf[1, pl.ds(j * L, L)] for j in range(GV))

                def row_body(r2, carry):
                    out = list(carry)
                    for dr in range(UNROLL):
                        r = r2 * UNROLL + dr
                        wlv = plsc.load_gather(
                            wlb, [jnp.full((L,), r, jnp.int32)])
                        for j in range(GV):
                            xv = xb[r, pl.ds(j * L, L)]
                            out[j] = out[j] + wlv * xv
                    return tuple(out)

                accs = lax.fori_loop(0, R // UNROLL, row_body, accs)
                for j in range(GV):
                    accbuf[1, pl.ds(j * L, L)] = accs[j]

            @pl.when(t0 + R <= len_s)
            def _():
                accs = tuple(accbuf[v, pl.ds(j * L, L)]
                             for v in range(2) for j in range(GV))

                def row_body(r2, carry):
                    out = list(carry)
                    for dr in range(UNROLL):
                        r = r2 * UNROLL + dr
                        wlv = plsc.load_gather(
                            wlb, [jnp.full((L,), r, jnp.int32)])
                        for j in range(GV):
                            xv = xb[r, pl.ds(j * L, L)]
                            out[j] = out[j] + xv
                            out[GV + j] = out[GV + j] + wlv * xv
                    return tuple(out)

                accs = lax.fori_loop(0, R // UNROLL, row_body, accs)
                for v in range(2):
                    for j in range(GV):
                        accbuf[v, pl.ds(j * L, L)] = accs[v * GV + j]

            @pl.when(jnp.logical_and(t0 < len_s, t0 + R > len_s))
            def _():
                accs = tuple(accbuf[v, pl.ds(j * L, L)]
                             for v in range(2) for j in range(GV))

                def row_body(r2, carry):
                    out = list(carry)
                    for dr in range(UNROLL):
                        r = r2 * UNROLL + dr
                        wlv = plsc.load_gather(
                            wlb, [jnp.full((L,), r, jnp.int32)])
                        rgv = jnp.full((L,), t0 + r, jnp.int32)
                        wgv = jnp.where(rgv < lenvi, ones, zeros)
                        for j in range(GV):
                            xv = xb[r, pl.ds(j * L, L)]
                            out[j] = out[j] + wgv * xv
                            out[GV + j] = out[GV + j] + wlv * xv
                    return tuple(out)

                accs = lax.fori_loop(0, R // UNROLL, row_body, accs)
                for v in range(2):
                    for j in range(GV):
                        accbuf[v, pl.ds(j * L, L)] = accs[v * GV + j]

            @pl.when(ci + 2 < NCHUNK)
            def _():
                x_copy(ci + 2, xb, sem).start()
                wl_copy(ci + 2, wlb, sem).start()
        return cnt

    cnt = lax.fori_loop(0, NCHUNK // 2, chunk2_body, zeros)

    deng = jnp.maximum(lenvi.astype(jnp.float32), ones)
    denl = jnp.maximum(jnp.full((L,), jnp.sum(cnt)), ones)
    for v, den in ((0, deng), (1, denl)):
        for j in range(GV):
            outstage[pl.ds(j * L, L)] = accbuf[v, pl.ds(j * L, L)] / den
        pltpu.sync_copy(outstage.at[pl.ds(0, HH)],
                        out_hbm.at[b, pl.ds(v * C_SC + c0, HH)])


def _tc_body(x_ref, w_ref, out_ref, acc_ref, den_ref):
    t = pl.program_id(1)
    w = x_ref.shape[2]

    @pl.when(t == 0)
    def _():
        acc_ref[...] = jnp.zeros_like(acc_ref)
        den_ref[0] = 0.0
        den_ref[1] = 0.0

    x = x_ref[0]
    w2 = w_ref[0, 0]
    acc_ref[...] += jnp.dot(w2, x, preferred_element_type=jnp.float32)
    den_ref[0] += jnp.sum(w2[0:1])
    den_ref[1] += jnp.sum(w2[1:2])

    @pl.when(t == NT - 1)
    def _():
        deng = jnp.maximum(den_ref[0], 1.0)
        denl = jnp.maximum(den_ref[1], 1.0)
        out_ref[...] = jnp.concatenate(
            [acc_ref[0:1] / deng, acc_ref[1:2] / denl], axis=0
        ).reshape(1, 2, w)


def _make_tc(nb, b_off, w, c_off):
    return pl.pallas_call(
        _tc_body,
        grid=(nb, NT),
        in_specs=[
            pl.BlockSpec((1, TBLK, w),
                         lambda bb, t: (bb + b_off, t, c_off // w)),
            pl.BlockSpec((1, 1, 2, TBLK), lambda bb, t: (bb + b_off, t, 0, 0)),
        ],
        out_specs=pl.BlockSpec((1, 2, w), lambda bb, t: (bb, 0, 0)),
        out_shape=jax.ShapeDtypeStruct((nb, 2, w), jnp.float32),
        scratch_shapes=[
            pltpu.VMEM((2, w), jnp.float32),
            pltpu.SMEM((2,), jnp.float32),
        ],
        compiler_params=pltpu.CompilerParams(
            dimension_semantics=("parallel", "arbitrary")),
    )


_pool_tc_hi = _make_tc(B - B_SC, B_SC, H, 0)


def kernel(inputs, lengths, mask):
    lens = lengths.astype(jnp.int32)
    wg = (jnp.arange(T, dtype=jnp.int32)[None, :]
          < lens[:, None]).astype(jnp.float32)
    wl = mask.astype(jnp.float32)
    w4 = jnp.stack([wg.reshape(B, NT, TBLK), wl.reshape(B, NT, TBLK)], axis=2)
    out_hi = _pool_tc_hi(inputs, w4)
    out_sc = _pool_sc(inputs, wl, lens)
    rows_hi = out_hi.reshape(B - B_SC, 2 * H)
    return jnp.concatenate([out_sc, rows_hi], axis=0)
